# baseline (speedup 1.0000x reference)
import functools

import jax
import jax.numpy as jnp
from jax.experimental import pallas as pl
from jax.experimental.pallas import tpu as pltpu

_TOP_K = 2


def _router_kernel(x_ref, w_ref, b_ref, logits_ref, probs_ref, idx_ref):
    j = pl.program_id(1)
    part = jnp.dot(x_ref[...], w_ref[...], preferred_element_type=jnp.float32)

    @pl.when(j == 0)
    def _first():
        logits_ref[...] = part + b_ref[...]

    @pl.when(j == 1)
    def _last():
        logits = logits_ref[...] + part
        logits_ref[...] = logits

        cols = jax.lax.broadcasted_iota(jnp.int32, logits.shape, 1)
        max1 = jnp.max(logits, axis=1, keepdims=True)
        idx1 = jnp.argmax(logits, axis=1)
        masked = jnp.where(cols == idx1[:, None], -jnp.inf, logits)
        max2 = jnp.max(masked, axis=1, keepdims=True)
        idx2 = jnp.argmax(masked, axis=1)

        e2 = jnp.exp(max2 - max1)
        denom = 1.0 + e2
        probs_ref[...] = jnp.concatenate([1.0 / denom, e2 / denom], axis=1)
        idx_ref[...] = jnp.stack([idx1, idx2], axis=1).astype(jnp.int32)


@functools.partial(jax.jit, static_argnames=("block_t",))
def _run(x, w_t, b2d, block_t):
    n_tokens, d_model = x.shape
    n_experts = w_t.shape[1]
    dh = d_model // 2
    grid = (n_tokens // block_t, 2)
    return pl.pallas_call(
        _router_kernel,
        grid=grid,
        compiler_params=pltpu.CompilerParams(
            dimension_semantics=("parallel", "arbitrary")),
        in_specs=[
            pl.BlockSpec((block_t, dh), lambda i, j: (i, j)),
            pl.BlockSpec((dh, n_experts), lambda i, j: (j, 0)),
            pl.BlockSpec((1, n_experts), lambda i, j: (0, 0)),
        ],
        out_specs=[
            pl.BlockSpec((block_t, n_experts), lambda i, j: (i, 0)),
            pl.BlockSpec((block_t, _TOP_K), lambda i, j: (i, 0)),
            pl.BlockSpec((block_t, _TOP_K), lambda i, j: (i, 0)),
        ],
        out_shape=[
            jax.ShapeDtypeStruct((n_tokens, n_experts), jnp.float32),
            jax.ShapeDtypeStruct((n_tokens, _TOP_K), jnp.float32),
            jax.ShapeDtypeStruct((n_tokens, _TOP_K), jnp.int32),
        ],
    )(x, w_t, b2d)


def kernel(x, W, b):
    logits, probs, idx = _run(x, W.T, b.reshape(1, -1), 1024)
    return (probs, idx, logits)


# speedup vs baseline: 1.3066x; 1.3066x over previous
import functools

import jax
import jax.numpy as jnp
from jax.experimental import pallas as pl
from jax.experimental.pallas import tpu as pltpu

_TOP_K = 2


def _router_kernel(x_ref, w_ref, b_ref, logits_ref, probs_ref, idx_ref):
    logits = (
        jax.lax.dot_general(
            x_ref[...], w_ref[...],
            dimension_numbers=(((1,), (1,)), ((), ())),
            preferred_element_type=jnp.float32)
        + b_ref[...]
    )
    logits_ref[...] = logits

    cols = jax.lax.broadcasted_iota(jnp.int32, logits.shape, 1)
    max1 = jnp.max(logits, axis=1, keepdims=True)
    idx1 = jnp.argmax(logits, axis=1)
    masked = jnp.where(cols == idx1[:, None], -jnp.inf, logits)
    max2 = jnp.max(masked, axis=1, keepdims=True)
    idx2 = jnp.argmax(masked, axis=1)

    e2 = jnp.exp(max2 - max1)
    denom = 1.0 + e2
    probs_ref[...] = jnp.concatenate([1.0 / denom, e2 / denom], axis=1)
    idx_ref[...] = jnp.stack([idx1, idx2], axis=1).astype(jnp.int32)


@functools.partial(jax.jit, static_argnames=("block_t",))
def _run(x, w, b2d, block_t):
    n_tokens, d_model = x.shape
    n_experts = w.shape[0]
    grid = (n_tokens // block_t,)
    return pl.pallas_call(
        _router_kernel,
        grid=grid,
        compiler_params=pltpu.CompilerParams(
            dimension_semantics=("parallel",)),
        in_specs=[
            pl.BlockSpec((block_t, d_model), lambda i: (i, 0)),
            pl.BlockSpec((n_experts, d_model), lambda i: (0, 0)),
            pl.BlockSpec((1, n_experts), lambda i: (0, 0)),
        ],
        out_specs=[
            pl.BlockSpec((block_t, n_experts), lambda i: (i, 0)),
            pl.BlockSpec((block_t, _TOP_K), lambda i: (i, 0)),
            pl.BlockSpec((block_t, _TOP_K), lambda i: (i, 0)),
        ],
        out_shape=[
            jax.ShapeDtypeStruct((n_tokens, n_experts), jnp.float32),
            jax.ShapeDtypeStruct((n_tokens, _TOP_K), jnp.float32),
            jax.ShapeDtypeStruct((n_tokens, _TOP_K), jnp.int32),
        ],
    )(x, w, b2d)


def kernel(x, W, b):
    logits, probs, idx = _run(x, W, b.reshape(1, -1), 1024)
    return (probs, idx, logits)
